# Initial kernel scaffold; baseline (speedup 1.0000x reference)
import jax
import jax.numpy as jnp
from jax.experimental import pallas as pl


def kernel(world_to_camera, projection, image_width, image_height, pixel_gaussian_idx, pixel_alpha, logits, codebooks):
    raise NotImplementedError("write your pallas kernel here")


# speedup vs baseline: 10.7457x; 10.7457x over previous
import functools

import jax
import jax.numpy as jnp
from jax import lax
from jax.experimental import pallas as pl
from jax.experimental.pallas import tpu as pltpu
from jax.experimental.pallas import tpu_sc as plsc

TOPK = 4


def _topk_softmax_body(x_ref, o_ref):
    x = x_ref[...]
    rows, cols = x.shape
    iota = lax.broadcasted_iota(jnp.int32, (rows, cols), 1)
    xm = x
    vals = []
    sel = []
    for _ in range(TOPK):
        m = jnp.max(xm, axis=1, keepdims=True)
        first = jnp.min(jnp.where(xm >= m, iota, cols), axis=1, keepdims=True)
        vals.append(m)
        sel.append(first)
        xm = jnp.where(iota == first, -jnp.inf, xm)
    m0 = vals[0]
    es = [jnp.exp(v - m0) for v in vals]
    z = es[0] + es[1] + es[2] + es[3]
    out = jnp.zeros_like(x)
    for i in range(TOPK):
        out = out + jnp.where(iota == sel[i], es[i] / z, 0.0)
    o_ref[...] = out


def _topk_softmax(logits, block_rows=2000):
    n, c = logits.shape
    grid = n // block_rows
    return pl.pallas_call(
        _topk_softmax_body,
        grid=(grid,),
        in_specs=[pl.BlockSpec((block_rows, c), lambda i: (i, 0))],
        out_specs=pl.BlockSpec((block_rows, c), lambda i: (i, 0)),
        out_shape=jax.ShapeDtypeStruct((n, c), jnp.float32),
    )(logits)


def _make_sc_gather(n_rows, d, total):
    info = plsc.get_sparse_core_info()
    nc, ns = info.num_cores, info.num_subcores
    nw = nc * ns
    per_w = total // nw
    ch = 128
    nstep = per_w // ch
    mesh = plsc.VectorSubcoreMesh(core_axis_name="c", subcore_axis_name="s")

    @functools.partial(
        pl.kernel,
        out_type=jax.ShapeDtypeStruct((total, d), jnp.float32),
        mesh=mesh,
        compiler_params=pltpu.CompilerParams(use_tc_tiling_on_sc=False),
        scratch_types=[
            pltpu.VMEM((ch,), jnp.int32),
            pltpu.VMEM((ch, d), jnp.float32),
            pltpu.SemaphoreType.DMA,
        ],
    )
    def sc_gather(table_hbm, idx_hbm, out_hbm, idx_v, rows_v, sem):
        wid = lax.axis_index("s") * nc + lax.axis_index("c")
        base = wid * per_w

        def body(i, carry):
            off = pl.multiple_of(base + i * ch, ch)
            pltpu.sync_copy(idx_hbm.at[pl.ds(off, ch)], idx_v)
            pltpu.async_copy(table_hbm.at[idx_v], rows_v, sem).wait()
            pltpu.sync_copy(rows_v, out_hbm.at[pl.ds(off, ch)])
            return carry

        lax.fori_loop(0, nstep, body, 0)

    return sc_gather


def _blend_matmul_body(k_hits, cb_dim, g_ref, a_ref, c_ref, f_ref, am_ref):
    g = g_ref[...]
    a = jnp.clip(a_ref[...], 0.0, 0.999)
    pb = g.shape[0]
    trans = jnp.ones((pb, 1), jnp.float32)
    wm = jnp.zeros((pb, cb_dim), jnp.float32)
    am = jnp.zeros((pb, 1), jnp.float32)
    for k in range(k_hits):
        ak = a[:, k:k + 1]
        bl = trans * ak
        am = am + bl
        wm = wm + bl * g[:, k * cb_dim:(k + 1) * cb_dim]
        trans = trans * (1.0 - ak)
    f_ref[...] = jnp.dot(wm, c_ref[...], preferred_element_type=jnp.float32)
    am_ref[...] = am


def _blend_matmul(gathered, alpha, codebook, block_px=512):
    p, kc = gathered.shape
    k_hits = alpha.shape[1]
    cb_dim = kc // k_hits
    clip_dims = codebook.shape[1]
    grid = p // block_px
    body = functools.partial(_blend_matmul_body, k_hits, cb_dim)
    return pl.pallas_call(
        body,
        grid=(grid,),
        in_specs=[
            pl.BlockSpec((block_px, kc), lambda i: (i, 0)),
            pl.BlockSpec((block_px, k_hits), lambda i: (i, 0)),
            pl.BlockSpec((cb_dim, clip_dims), lambda i: (0, 0)),
        ],
        out_specs=[
            pl.BlockSpec((block_px, clip_dims), lambda i: (i, 0)),
            pl.BlockSpec((block_px, 1), lambda i: (i, 0)),
        ],
        out_shape=[
            jax.ShapeDtypeStruct((p, clip_dims), jnp.float32),
            jax.ShapeDtypeStruct((p, 1), jnp.float32),
        ],
    )(gathered, alpha, codebook)


def kernel(world_to_camera, projection, image_width, image_height,
           pixel_gaussian_idx, pixel_alpha, logits, codebooks):
    n, cb_dim = logits.shape
    bz, h, w, k_hits = pixel_alpha.shape
    clip_dims = codebooks.shape[2]
    p = bz * h * w

    idx = pixel_gaussian_idx.reshape(p * k_hits).astype(jnp.int32)
    alpha = pixel_alpha.reshape(p, k_hits)

    weights = _topk_softmax(logits)
    gathered = _make_sc_gather(n, cb_dim, p * k_hits)(weights, idx)
    feature, alpha_map = _blend_matmul(
        gathered.reshape(p, k_hits * cb_dim), alpha, codebooks[0])
    return (feature.reshape(bz, h, w, clip_dims),
            alpha_map.reshape(bz, h, w, 1))


# speedup vs baseline: 15.2219x; 1.4166x over previous
import functools

import jax
import jax.numpy as jnp
from jax import lax
from jax.experimental import pallas as pl
from jax.experimental.pallas import tpu as pltpu
from jax.experimental.pallas import tpu_sc as plsc

TOPK = 4

_SC_PARAMS = pltpu.CompilerParams(use_tc_tiling_on_sc=False,
                                  needs_layout_passes=False)


def _topk_pack_body(x_ref, o_ref):
    x = x_ref[...]
    rows, cols = x.shape
    iota = lax.broadcasted_iota(jnp.int32, (rows, cols), 1)
    xb = lax.bitcast_convert_type(x, jnp.int32)
    tie = jnp.where(xb >= 0, (cols - 1) - iota, iota)
    key = lax.bitcast_convert_type((xb & ~63) | tie, jnp.float32)
    ms = []
    for _ in range(TOPK):
        m = jnp.max(key, axis=1, keepdims=True)
        key = jnp.where(key == m, -jnp.inf, key)
        ms.append(m)
    m_cat = jnp.concatenate(ms, axis=1)
    mb = lax.bitcast_convert_type(m_cat, jnp.int32)
    low = mb & 63
    lanes = jnp.where(mb >= 0, (cols - 1) - low, low)
    e = jnp.exp(m_cat - ms[0])
    soft = e / jnp.sum(e, axis=1, keepdims=True)
    packed = jnp.concatenate([soft, lanes.astype(jnp.float32)], axis=1)
    o_ref[...] = packed.T


def _topk_pack(logits, block_rows=2048):
    n, c = logits.shape
    grid = -(-n // block_rows)
    n_pad = grid * block_rows
    return pl.pallas_call(
        _topk_pack_body,
        grid=(grid,),
        in_specs=[pl.BlockSpec((block_rows, c), lambda i: (i, 0))],
        out_specs=pl.BlockSpec((2 * TOPK, block_rows), lambda i: (0, i)),
        out_shape=jax.ShapeDtypeStruct((2 * TOPK, n_pad), jnp.float32),
    )(logits)


def _make_sc_repack(n_pad, nfields):
    info = plsc.get_sparse_core_info()
    nw = info.num_cores * info.num_subcores
    per_w = n_pad // nw
    mesh = plsc.VectorSubcoreMesh(core_axis_name="c", subcore_axis_name="s")

    @functools.partial(
        pl.kernel,
        out_type=jax.ShapeDtypeStruct((n_pad * nfields,), jnp.float32),
        mesh=mesh,
        compiler_params=_SC_PARAMS,
        scratch_types=[
            pltpu.VMEM((nfields * per_w,), jnp.float32),
            pltpu.VMEM((nfields * per_w,), jnp.float32),
        ],
    )
    def sc_repack(fm_hbm, aos_hbm, buf_in, buf_out):
        wid = lax.axis_index("s") * info.num_cores + lax.axis_index("c")
        g0 = wid * per_w
        for f in range(nfields):
            pltpu.sync_copy(fm_hbm.at[pl.ds(f * n_pad + g0, per_w)],
                            buf_in.at[pl.ds(f * per_w, per_w)])

        def body(i, carry):
            lanes = lax.broadcasted_iota(jnp.int32, (16,), 0)
            s_local = jnp.full((16,), i * 2, jnp.int32) + lanes // nfields
            src = (lanes % nfields) * per_w + s_local
            rec = plsc.load_gather(buf_in, [src])
            buf_out[pl.ds(i * 16, 16)] = rec
            return carry

        lax.fori_loop(0, nfields * per_w // 16, body, 0)
        pltpu.sync_copy(buf_out, aos_hbm.at[pl.ds(g0 * nfields,
                                                  nfields * per_w)])

    return sc_repack


def _blend_body(k_hits, a_ref, b_ref, am_ref):
    a = jnp.clip(a_ref[...], 0.0, 0.999)
    rows = a.shape[0]
    trans = jnp.ones((rows, 1), jnp.float32)
    am = jnp.zeros((rows, 1), jnp.float32)
    bls = []
    for k in range(k_hits):
        ak = a[:, k:k + 1]
        bl = trans * ak
        am = am + bl
        bls.append(bl)
        trans = trans * (1.0 - ak)
    blend = jnp.concatenate(bls, axis=1)
    b_ref[...] = blend.T
    am_ref[...] = am


def _blend_tc(alpha, block_px=2048):
    p, k_hits = alpha.shape
    grid = p // block_px
    body = functools.partial(_blend_body, k_hits)
    return pl.pallas_call(
        body,
        grid=(grid,),
        in_specs=[pl.BlockSpec((block_px, k_hits), lambda i: (i, 0))],
        out_specs=[
            pl.BlockSpec((k_hits, block_px), lambda i: (0, i)),
            pl.BlockSpec((block_px, 1), lambda i: (i, 0)),
        ],
        out_shape=[
            jax.ShapeDtypeStruct((k_hits, p), jnp.float32),
            jax.ShapeDtypeStruct((p, 1), jnp.float32),
        ],
    )(alpha)


def _make_sc_reduce(n_pad, total_px, cb_dim, k_hits):
    info = plsc.get_sparse_core_info()
    nc = info.num_cores
    nw = nc * info.num_subcores
    chunk_px = 256
    chunk_slots = chunk_px * k_hits
    px_per_w = total_px // nw
    nchunk = px_per_w // chunk_px
    idx_rows = chunk_slots // 128
    wm_words = chunk_px * cb_dim
    mesh = plsc.VectorSubcoreMesh(core_axis_name="c", subcore_axis_name="s")

    @functools.partial(
        pl.kernel,
        out_type=jax.ShapeDtypeStruct((total_px * cb_dim,), jnp.float32),
        mesh=mesh,
        compiler_params=_SC_PARAMS,
        scratch_types=[
            pltpu.VMEM((idx_rows, 128), jnp.int32),
            pltpu.VMEM((chunk_slots, k_hits), jnp.float32),
            pltpu.VMEM((k_hits * chunk_px,), jnp.float32),
            pltpu.VMEM((wm_words,), jnp.float32),
            pltpu.SemaphoreType.DMA,
        ],
    )
    def sc_reduce(aos_hbm, idx_hbm, blt_hbm, out_hbm,
                  idx_v, pk_v, bl_v, wm_v, sem):
        wid = lax.axis_index("s") * nc + lax.axis_index("c")

        def chunk_body(c, carry):
            r0 = wid * (idx_rows * nchunk) + c * idx_rows
            p0 = wid * px_per_w + c * chunk_px
            w0 = wid * (wm_words * nchunk) + c * wm_words
            pltpu.sync_copy(idx_hbm.at[pl.ds(r0, idx_rows)], idx_v)
            for k in range(k_hits):
                pltpu.sync_copy(
                    blt_hbm.at[pl.ds(k * total_px + p0, chunk_px)],
                    bl_v.at[pl.ds(k * chunk_px, chunk_px)])
            copies = []
            for j in range(idx_rows):
                copies.append(pltpu.async_copy(
                    aos_hbm.at[idx_v.at[j]],
                    pk_v.at[pl.ds(j * 128, 128)], sem))
            for cp in copies:
                cp.wait()

            def zero_body(i, zc):
                wm_v[pl.ds(i * 16, 16)] = jnp.zeros((16,), jnp.float32)
                return zc
            lax.fori_loop(0, wm_words // 16, zero_body, 0)

            def px_body(p, pc):
                lanes = lax.broadcasted_iota(jnp.int32, (16,), 0)
                rowpat = lanes // 4
                colpat = lanes % 4
                base = jnp.full((16,), p * cb_dim, jnp.int32)
                sp = jnp.full((16,), p * k_hits, jnp.int32)
                blp = jnp.full((16,), p, jnp.int32) + rowpat * chunk_px
                for half in range(2):
                    rows = sp + rowpat + (half * 4)
                    vals = plsc.load_gather(pk_v, [rows, colpat])
                    idxf = plsc.load_gather(pk_v, [rows, colpat + 4])
                    bl = plsc.load_gather(
                        bl_v, [blp + (half * 4 * chunk_px)])
                    tgt = idxf.astype(jnp.int32) + base
                    plsc.addupdate_scatter(wm_v, [tgt], vals * bl)
                return pc
            lax.fori_loop(0, chunk_px, px_body, 0)

            pltpu.sync_copy(wm_v, out_hbm.at[pl.ds(w0, wm_words)])
            return carry

        lax.fori_loop(0, nchunk, chunk_body, 0)

    return sc_reduce


def _decode_body(w_ref, c_ref, f_ref):
    f_ref[...] = jnp.dot(w_ref[...], c_ref[...],
                         preferred_element_type=jnp.float32)


def _decode_matmul(wm, codebook, block_px=512):
    p, cb_dim = wm.shape
    clip_dims = codebook.shape[1]
    grid = p // block_px
    return pl.pallas_call(
        _decode_body,
        grid=(grid,),
        in_specs=[
            pl.BlockSpec((block_px, cb_dim), lambda i: (i, 0)),
            pl.BlockSpec((cb_dim, clip_dims), lambda i: (0, 0)),
        ],
        out_specs=pl.BlockSpec((block_px, clip_dims), lambda i: (i, 0)),
        out_shape=jax.ShapeDtypeStruct((p, clip_dims), jnp.float32),
    )(wm, codebook)


def kernel(world_to_camera, projection, image_width, image_height,
           pixel_gaussian_idx, pixel_alpha, logits, codebooks):
    n, cb_dim = logits.shape
    bz, h, w, k_hits = pixel_alpha.shape
    clip_dims = codebooks.shape[2]
    p = bz * h * w
    total_slots = p * k_hits
    nfields = 2 * TOPK

    idx2d = pixel_gaussian_idx.reshape(total_slots // 128, 128).astype(jnp.int32)
    alpha = pixel_alpha.reshape(p, k_hits)

    fm = _topk_pack(logits)
    n_pad = fm.shape[1]
    aos1d = _make_sc_repack(n_pad, nfields)(fm.reshape(nfields * n_pad))
    blt, alpha_map = _blend_tc(alpha)
    wm1d = _make_sc_reduce(n_pad, p, cb_dim, k_hits)(
        aos1d.reshape(n_pad, nfields), idx2d, blt.reshape(k_hits * p))
    feature = _decode_matmul(wm1d.reshape(p, cb_dim), codebooks[0])
    return (feature.reshape(bz, h, w, clip_dims),
            alpha_map.reshape(bz, h, w, 1))


# speedup vs baseline: 22.9385x; 1.5069x over previous
import functools

import jax
import jax.numpy as jnp
from jax import lax
from jax.experimental import pallas as pl
from jax.experimental.pallas import tpu as pltpu
from jax.experimental.pallas import tpu_sc as plsc

TOPK = 4

_SC_PARAMS = pltpu.CompilerParams(use_tc_tiling_on_sc=False,
                                  needs_layout_passes=False)


def _topk_pack_body(x_ref, o_ref):
    x = x_ref[...]
    rows, cols = x.shape
    xt = x.T
    iota = lax.broadcasted_iota(jnp.int32, (cols, rows), 0)
    xb = lax.bitcast_convert_type(xt, jnp.int32)
    tie = jnp.where(xb >= 0, (cols - 1) - iota, iota)
    key = lax.bitcast_convert_type((xb & ~63) | tie, jnp.float32)
    ms = []
    for _ in range(TOPK):
        m = jnp.max(key, axis=0, keepdims=True)
        key = jnp.where(key == m, -jnp.inf, key)
        ms.append(m)
    m_cat = jnp.concatenate(ms, axis=0)
    mb = lax.bitcast_convert_type(m_cat, jnp.int32)
    low = mb & 63
    lanes = jnp.where(mb >= 0, (cols - 1) - low, low)
    e = jnp.exp(m_cat - ms[0])
    soft = e / jnp.sum(e, axis=0, keepdims=True)
    o_ref[...] = jnp.concatenate([soft, lanes.astype(jnp.float32)], axis=0)


def _topk_pack(logits, block_rows=2048):
    n, c = logits.shape
    grid = -(-n // block_rows)
    n_pad = grid * block_rows
    return pl.pallas_call(
        _topk_pack_body,
        grid=(grid,),
        in_specs=[pl.BlockSpec((block_rows, c), lambda i: (i, 0))],
        out_specs=pl.BlockSpec((2 * TOPK, block_rows), lambda i: (0, i)),
        out_shape=jax.ShapeDtypeStruct((2 * TOPK, n_pad), jnp.float32),
    )(logits)


def _make_sc_repack(n_pad, nfields):
    info = plsc.get_sparse_core_info()
    nw = info.num_cores * info.num_subcores
    per_w = n_pad // nw
    mesh = plsc.VectorSubcoreMesh(core_axis_name="c", subcore_axis_name="s")

    @functools.partial(
        pl.kernel,
        out_type=jax.ShapeDtypeStruct((n_pad * nfields,), jnp.float32),
        mesh=mesh,
        compiler_params=_SC_PARAMS,
        scratch_types=[
            pltpu.VMEM((nfields * per_w,), jnp.float32),
            pltpu.VMEM((nfields * per_w,), jnp.float32),
        ],
    )
    def sc_repack(fm_hbm, aos_hbm, buf_in, buf_out):
        wid = lax.axis_index("s") * info.num_cores + lax.axis_index("c")
        g0 = wid * per_w
        for f in range(nfields):
            pltpu.sync_copy(fm_hbm.at[pl.ds(f * n_pad + g0, per_w)],
                            buf_in.at[pl.ds(f * per_w, per_w)])

        def body(i, carry):
            lanes = lax.broadcasted_iota(jnp.int32, (16,), 0)
            s_local = jnp.full((16,), i * 2, jnp.int32) + lanes // nfields
            src = (lanes % nfields) * per_w + s_local
            rec = plsc.load_gather(buf_in, [src])
            buf_out[pl.ds(i * 16, 16)] = rec
            return carry

        lax.fori_loop(0, nfields * per_w // 16, body, 0)
        pltpu.sync_copy(buf_out, aos_hbm.at[pl.ds(g0 * nfields,
                                                  nfields * per_w)])

    return sc_repack


def _blend_body(k_hits, a_ref, b_ref, am_ref):
    at = jnp.clip(a_ref[...].T, 0.0, 0.999)
    rows = at.shape[1]
    trans = jnp.ones((1, rows), jnp.float32)
    bls = []
    for k in range(k_hits):
        ak = at[k:k + 1, :]
        bls.append(trans * ak)
        trans = trans * (1.0 - ak)
    blend = jnp.concatenate(bls, axis=0)
    b_ref[...] = blend
    am_ref[...] = jnp.sum(blend, axis=0, keepdims=True)[None]


def _blend_tc(alpha, block_px=2048):
    p, k_hits = alpha.shape
    grid = p // block_px
    body = functools.partial(_blend_body, k_hits)
    return pl.pallas_call(
        body,
        grid=(grid,),
        in_specs=[pl.BlockSpec((block_px, k_hits), lambda i: (i, 0))],
        out_specs=[
            pl.BlockSpec((k_hits, block_px), lambda i: (0, i)),
            pl.BlockSpec((1, 1, block_px), lambda i: (i, 0, 0)),
        ],
        out_shape=[
            jax.ShapeDtypeStruct((k_hits, p), jnp.float32),
            jax.ShapeDtypeStruct((grid, 1, block_px), jnp.float32),
        ],
    )(alpha)


def _make_sc_reduce(n_pad, total_px, cb_dim, k_hits):
    info = plsc.get_sparse_core_info()
    nc = info.num_cores
    nw = nc * info.num_subcores
    chunk_px = 256
    chunk_slots = chunk_px * k_hits
    px_per_w = total_px // nw
    nchunk = px_per_w // chunk_px
    idx_rows = chunk_slots // 128
    wm_words = chunk_px * cb_dim
    mesh = plsc.VectorSubcoreMesh(core_axis_name="c", subcore_axis_name="s")

    @functools.partial(
        pl.kernel,
        out_type=jax.ShapeDtypeStruct((total_px * cb_dim,), jnp.float32),
        mesh=mesh,
        compiler_params=_SC_PARAMS,
        scratch_types=[
            pltpu.VMEM((idx_rows, 128), jnp.int32),
            pltpu.VMEM((chunk_slots, k_hits), jnp.float32),
            pltpu.VMEM((k_hits * chunk_px,), jnp.float32),
            pltpu.VMEM((wm_words,), jnp.float32),
            pltpu.SemaphoreType.DMA,
        ],
    )
    def sc_reduce(aos_hbm, idx_hbm, blt_hbm, out_hbm,
                  idx_v, pk_v, bl_v, wm_v, sem):
        wid = lax.axis_index("s") * nc + lax.axis_index("c")

        def chunk_body(c, carry):
            r0 = wid * (idx_rows * nchunk) + c * idx_rows
            p0 = wid * px_per_w + c * chunk_px
            w0 = wid * (wm_words * nchunk) + c * wm_words
            pltpu.sync_copy(idx_hbm.at[pl.ds(r0, idx_rows)], idx_v)
            for k in range(k_hits):
                pltpu.sync_copy(
                    blt_hbm.at[pl.ds(k * total_px + p0, chunk_px)],
                    bl_v.at[pl.ds(k * chunk_px, chunk_px)])
            copies = []
            for j in range(idx_rows):
                copies.append(pltpu.async_copy(
                    aos_hbm.at[idx_v.at[j]],
                    pk_v.at[pl.ds(j * 128, 128)], sem))
            for cp in copies:
                cp.wait()

            def zero_body(i, zc):
                wm_v[pl.ds(i * 16, 16)] = jnp.zeros((16,), jnp.float32)
                return zc
            lax.fori_loop(0, wm_words // 16, zero_body, 0)

            def px_body(p, pc):
                lanes = lax.broadcasted_iota(jnp.int32, (16,), 0)
                rowpat = lanes // 4
                colpat = lanes % 4
                base = jnp.full((16,), p * cb_dim, jnp.int32)
                sp = jnp.full((16,), p * k_hits, jnp.int32)
                blp = jnp.full((16,), p, jnp.int32) + rowpat * chunk_px
                for half in range(2):
                    rows = sp + rowpat + (half * 4)
                    vals = plsc.load_gather(pk_v, [rows, colpat])
                    idxf = plsc.load_gather(pk_v, [rows, colpat + 4])
                    bl = plsc.load_gather(
                        bl_v, [blp + (half * 4 * chunk_px)])
                    tgt = idxf.astype(jnp.int32) + base
                    plsc.addupdate_scatter(wm_v, [tgt], vals * bl)
                return pc
            lax.fori_loop(0, chunk_px, px_body, 0)

            pltpu.sync_copy(wm_v, out_hbm.at[pl.ds(w0, wm_words)])
            return carry

        lax.fori_loop(0, nchunk, chunk_body, 0)

    return sc_reduce


def _decode_body(w_ref, c_ref, f_ref):
    f_ref[...] = jnp.dot(w_ref[...], c_ref[...],
                         preferred_element_type=jnp.float32)


def _decode_matmul(wm, codebook, block_px=512):
    p, cb_dim = wm.shape
    clip_dims = codebook.shape[1]
    grid = p // block_px
    return pl.pallas_call(
        _decode_body,
        grid=(grid,),
        in_specs=[
            pl.BlockSpec((block_px, cb_dim), lambda i: (i, 0)),
            pl.BlockSpec((cb_dim, clip_dims), lambda i: (0, 0)),
        ],
        out_specs=pl.BlockSpec((block_px, clip_dims), lambda i: (i, 0)),
        out_shape=jax.ShapeDtypeStruct((p, clip_dims), jnp.float32),
    )(wm, codebook)


def kernel(world_to_camera, projection, image_width, image_height,
           pixel_gaussian_idx, pixel_alpha, logits, codebooks):
    n, cb_dim = logits.shape
    bz, h, w, k_hits = pixel_alpha.shape
    clip_dims = codebooks.shape[2]
    p = bz * h * w
    total_slots = p * k_hits
    nfields = 2 * TOPK

    idx2d = pixel_gaussian_idx.reshape(total_slots // 128, 128).astype(jnp.int32)
    alpha = pixel_alpha.reshape(p, k_hits)

    fm = _topk_pack(logits)
    n_pad = fm.shape[1]
    aos1d = _make_sc_repack(n_pad, nfields)(fm.reshape(nfields * n_pad))
    blt, alpha_map = _blend_tc(alpha)
    wm1d = _make_sc_reduce(n_pad, p, cb_dim, k_hits)(
        aos1d.reshape(n_pad, nfields), idx2d, blt.reshape(k_hits * p))
    feature = _decode_matmul(wm1d.reshape(p, cb_dim), codebooks[0])
    return (feature.reshape(bz, h, w, clip_dims),
            alpha_map.reshape(bz, h, w, 1))


# speedup vs baseline: 25.7077x; 1.1207x over previous
import functools

import jax
import jax.numpy as jnp
from jax import lax
from jax.experimental import pallas as pl
from jax.experimental.pallas import tpu as pltpu
from jax.experimental.pallas import tpu_sc as plsc

TOPK = 4

_SC_PARAMS = pltpu.CompilerParams(use_tc_tiling_on_sc=False,
                                  needs_layout_passes=False)


def _topk_pack_body(x_ref, o_ref):
    x = x_ref[...]
    rows, cols = x.shape
    xt = x.T
    iota = lax.broadcasted_iota(jnp.int32, (cols, rows), 0)
    xb = lax.bitcast_convert_type(xt, jnp.int32)
    tie = jnp.where(xb >= 0, (cols - 1) - iota, iota)
    key = lax.bitcast_convert_type((xb & ~63) | tie, jnp.float32)
    ms = []
    for _ in range(TOPK):
        m = jnp.max(key, axis=0, keepdims=True)
        key = jnp.where(key == m, -jnp.inf, key)
        ms.append(m)
    m_cat = jnp.concatenate(ms, axis=0)
    mb = lax.bitcast_convert_type(m_cat, jnp.int32)
    low = mb & 63
    lanes = jnp.where(mb >= 0, (cols - 1) - low, low)
    e = jnp.exp(m_cat - ms[0])
    soft = e / jnp.sum(e, axis=0, keepdims=True)
    o_ref[...] = jnp.concatenate([soft, lanes.astype(jnp.float32)], axis=0)


def _topk_pack(logits, block_rows=2048):
    n, c = logits.shape
    grid = -(-n // block_rows)
    n_pad = grid * block_rows
    return pl.pallas_call(
        _topk_pack_body,
        grid=(grid,),
        in_specs=[pl.BlockSpec((block_rows, c), lambda i: (i, 0))],
        out_specs=pl.BlockSpec((2 * TOPK, block_rows), lambda i: (0, i)),
        out_shape=jax.ShapeDtypeStruct((2 * TOPK, n_pad), jnp.float32),
    )(logits)


def _make_sc_repack(n_pad, nfields):
    info = plsc.get_sparse_core_info()
    nw = info.num_cores * info.num_subcores
    per_w = n_pad // nw
    mesh = plsc.VectorSubcoreMesh(core_axis_name="c", subcore_axis_name="s")

    @functools.partial(
        pl.kernel,
        out_type=jax.ShapeDtypeStruct((n_pad * nfields,), jnp.float32),
        mesh=mesh,
        compiler_params=_SC_PARAMS,
        scratch_types=[
            pltpu.VMEM((nfields * per_w,), jnp.float32),
            pltpu.VMEM((nfields * per_w,), jnp.float32),
        ],
    )
    def sc_repack(fm_hbm, aos_hbm, buf_in, buf_out):
        wid = lax.axis_index("s") * info.num_cores + lax.axis_index("c")
        g0 = wid * per_w
        for f in range(nfields):
            pltpu.sync_copy(fm_hbm.at[pl.ds(f * n_pad + g0, per_w)],
                            buf_in.at[pl.ds(f * per_w, per_w)])

        def body(i, carry):
            lanes = lax.broadcasted_iota(jnp.int32, (16,), 0)
            s_local = jnp.full((16,), i * 2, jnp.int32) + lanes // nfields
            src = (lanes % nfields) * per_w + s_local
            rec = plsc.load_gather(buf_in, [src])
            buf_out[pl.ds(i * 16, 16)] = rec
            return carry

        lax.fori_loop(0, nfields * per_w // 16, body, 0)
        pltpu.sync_copy(buf_out, aos_hbm.at[pl.ds(g0 * nfields,
                                                  nfields * per_w)])

    return sc_repack


def _blend_body(k_hits, a_ref, b_ref, am_ref):
    at = jnp.clip(a_ref[...].T, 0.0, 0.999)
    rows = at.shape[1]
    trans = jnp.ones((1, rows), jnp.float32)
    bls = []
    for k in range(k_hits):
        ak = at[k:k + 1, :]
        bls.append(trans * ak)
        trans = trans * (1.0 - ak)
    blend = jnp.concatenate(bls, axis=0)
    b_ref[...] = blend
    am_ref[...] = jnp.sum(blend, axis=0, keepdims=True)[None]


def _blend_tc(alpha, block_px=2048):
    p, k_hits = alpha.shape
    grid = p // block_px
    body = functools.partial(_blend_body, k_hits)
    return pl.pallas_call(
        body,
        grid=(grid,),
        in_specs=[pl.BlockSpec((block_px, k_hits), lambda i: (i, 0))],
        out_specs=[
            pl.BlockSpec((k_hits, block_px), lambda i: (0, i)),
            pl.BlockSpec((1, 1, block_px), lambda i: (i, 0, 0)),
        ],
        out_shape=[
            jax.ShapeDtypeStruct((k_hits, p), jnp.float32),
            jax.ShapeDtypeStruct((grid, 1, block_px), jnp.float32),
        ],
    )(alpha)


def _make_sc_reduce(n_pad, total_px, cb_dim, k_hits):
    info = plsc.get_sparse_core_info()
    nc = info.num_cores
    nw = nc * info.num_subcores
    chunk_px = 256
    chunk_slots = chunk_px * k_hits
    px_per_w = total_px // nw
    nchunk = px_per_w // chunk_px
    idx_rows = chunk_slots // 128
    wm_words = chunk_px * cb_dim
    mesh = plsc.VectorSubcoreMesh(core_axis_name="c", subcore_axis_name="s")

    @functools.partial(
        pl.kernel,
        out_type=jax.ShapeDtypeStruct((total_px * cb_dim,), jnp.float32),
        mesh=mesh,
        compiler_params=_SC_PARAMS,
        scratch_types=[
            pltpu.VMEM((idx_rows, 128), jnp.int32),
            pltpu.VMEM((chunk_slots, k_hits), jnp.float32),
            pltpu.VMEM((k_hits * chunk_px,), jnp.float32),
            pltpu.VMEM((wm_words,), jnp.float32),
            pltpu.SemaphoreType.DMA,
            pltpu.SemaphoreType.DMA,
        ],
    )
    def sc_reduce(aos_hbm, idx_hbm, blt_hbm, out_hbm,
                  idx_v, pk_v, bl_v, wm_v, sem, sem_out):
        wid = lax.axis_index("s") * nc + lax.axis_index("c")
        lanes = lax.broadcasted_iota(jnp.int32, (16,), 0)
        rowpat = lanes // 4
        colpat = lanes % 4
        blpat = rowpat * chunk_px
        zeros16 = jnp.zeros((16,), jnp.float32)
        unroll = 4

        prev_out = None
        for c in range(nchunk):
            r0 = wid * (idx_rows * nchunk) + c * idx_rows
            p0 = wid * px_per_w + c * chunk_px
            w0 = wid * (wm_words * nchunk) + c * wm_words
            pltpu.sync_copy(idx_hbm.at[pl.ds(r0, idx_rows)], idx_v)
            copies = []
            for j in range(idx_rows):
                copies.append(pltpu.async_copy(
                    aos_hbm.at[idx_v.at[j]],
                    pk_v.at[pl.ds(j * 128, 128)], sem))
            for k in range(k_hits):
                copies.append(pltpu.async_copy(
                    blt_hbm.at[pl.ds(k * total_px + p0, chunk_px)],
                    bl_v.at[pl.ds(k * chunk_px, chunk_px)], sem))
            if prev_out is not None:
                prev_out.wait()

            def zero_body(i, zc):
                for t in range(8):
                    wm_v[pl.ds(i * 128 + t * 16, 16)] = zeros16
                return zc
            lax.fori_loop(0, wm_words // 128, zero_body, 0)
            for cp in copies:
                cp.wait()

            def px_body(i, pc):
                for u in range(unroll):
                    p = i * unroll + u
                    base = jnp.full((16,), p * cb_dim, jnp.int32)
                    sp = jnp.full((16,), p * k_hits, jnp.int32) + rowpat
                    blp = jnp.full((16,), p, jnp.int32) + blpat
                    for half in range(2):
                        rows = sp + (half * 4)
                        vals = plsc.load_gather(pk_v, [rows, colpat])
                        idxf = plsc.load_gather(pk_v, [rows, colpat + 4])
                        bl = plsc.load_gather(
                            bl_v, [blp + (half * 4 * chunk_px)])
                        tgt = idxf.astype(jnp.int32) + base
                        plsc.addupdate_scatter(wm_v, [tgt], vals * bl)
                return pc
            lax.fori_loop(0, chunk_px // unroll, px_body, 0)

            prev_out = pltpu.async_copy(
                wm_v, out_hbm.at[pl.ds(w0, wm_words)], sem_out)
        prev_out.wait()

    return sc_reduce


def _decode_body(w_ref, c_ref, f_ref):
    f_ref[...] = jnp.dot(w_ref[...], c_ref[...],
                         preferred_element_type=jnp.float32)


def _decode_matmul(wm, codebook, block_px=512):
    p, cb_dim = wm.shape
    clip_dims = codebook.shape[1]
    grid = p // block_px
    return pl.pallas_call(
        _decode_body,
        grid=(grid,),
        in_specs=[
            pl.BlockSpec((block_px, cb_dim), lambda i: (i, 0)),
            pl.BlockSpec((cb_dim, clip_dims), lambda i: (0, 0)),
        ],
        out_specs=pl.BlockSpec((block_px, clip_dims), lambda i: (i, 0)),
        out_shape=jax.ShapeDtypeStruct((p, clip_dims), jnp.float32),
    )(wm, codebook)


def kernel(world_to_camera, projection, image_width, image_height,
           pixel_gaussian_idx, pixel_alpha, logits, codebooks):
    n, cb_dim = logits.shape
    bz, h, w, k_hits = pixel_alpha.shape
    clip_dims = codebooks.shape[2]
    p = bz * h * w
    total_slots = p * k_hits
    nfields = 2 * TOPK

    idx2d = pixel_gaussian_idx.reshape(total_slots // 128, 128).astype(jnp.int32)
    alpha = pixel_alpha.reshape(p, k_hits)

    fm = _topk_pack(logits)
    n_pad = fm.shape[1]
    aos1d = _make_sc_repack(n_pad, nfields)(fm.reshape(nfields * n_pad))
    blt, alpha_map = _blend_tc(alpha)
    wm1d = _make_sc_reduce(n_pad, p, cb_dim, k_hits)(
        aos1d.reshape(n_pad, nfields), idx2d, blt.reshape(k_hits * p))
    feature = _decode_matmul(wm1d.reshape(p, cb_dim), codebooks[0])
    return (feature.reshape(bz, h, w, clip_dims),
            alpha_map.reshape(bz, h, w, 1))


# speedup vs baseline: 26.4512x; 1.0289x over previous
import functools

import jax
import jax.numpy as jnp
from jax import lax
from jax.experimental import pallas as pl
from jax.experimental.pallas import tpu as pltpu
from jax.experimental.pallas import tpu_sc as plsc

TOPK = 4

_SC_PARAMS = pltpu.CompilerParams(use_tc_tiling_on_sc=False,
                                  needs_layout_passes=False)


def _topk_pack_body(x_ref, o_ref):
    x = x_ref[...]
    rows, cols = x.shape
    xt = x.T
    iota = lax.broadcasted_iota(jnp.int32, (cols, rows), 0)
    xb = lax.bitcast_convert_type(xt, jnp.int32)
    tie = jnp.where(xb >= 0, (cols - 1) - iota, iota)
    key = lax.bitcast_convert_type((xb & ~63) | tie, jnp.float32)
    ms = []
    for _ in range(TOPK):
        m = jnp.max(key, axis=0, keepdims=True)
        key = jnp.where(key == m, -jnp.inf, key)
        ms.append(m)
    m_cat = jnp.concatenate(ms, axis=0)
    mb = lax.bitcast_convert_type(m_cat, jnp.int32)
    low = mb & 63
    lanes = jnp.where(mb >= 0, (cols - 1) - low, low)
    e = jnp.exp(m_cat - ms[0])
    soft = e / jnp.sum(e, axis=0, keepdims=True)
    o_ref[...] = jnp.concatenate([soft, lanes.astype(jnp.float32)], axis=0)


def _topk_pack(logits, block_rows=2048):
    n, c = logits.shape
    grid = -(-n // block_rows)
    n_pad = grid * block_rows
    return pl.pallas_call(
        _topk_pack_body,
        grid=(grid,),
        in_specs=[pl.BlockSpec((block_rows, c), lambda i: (i, 0))],
        out_specs=pl.BlockSpec((2 * TOPK, block_rows), lambda i: (0, i)),
        out_shape=jax.ShapeDtypeStruct((2 * TOPK, n_pad), jnp.float32),
    )(logits)


def _make_sc_repack(n_pad, nfields):
    info = plsc.get_sparse_core_info()
    nw = info.num_cores * info.num_subcores
    per_w = n_pad // nw
    mesh = plsc.VectorSubcoreMesh(core_axis_name="c", subcore_axis_name="s")

    @functools.partial(
        pl.kernel,
        out_type=jax.ShapeDtypeStruct((n_pad * nfields,), jnp.float32),
        mesh=mesh,
        compiler_params=_SC_PARAMS,
        scratch_types=[
            pltpu.VMEM((nfields * per_w,), jnp.float32),
            pltpu.VMEM((nfields * per_w,), jnp.float32),
        ],
    )
    def sc_repack(fm_hbm, aos_hbm, buf_in, buf_out):
        wid = lax.axis_index("s") * info.num_cores + lax.axis_index("c")
        g0 = wid * per_w
        for f in range(nfields):
            pltpu.sync_copy(fm_hbm.at[pl.ds(f * n_pad + g0, per_w)],
                            buf_in.at[pl.ds(f * per_w, per_w)])

        lanes = lax.broadcasted_iota(jnp.int32, (16,), 0)
        base_src = (lanes % nfields) * per_w + lanes // nfields

        def body(i, carry):
            for u in range(4):
                src = base_src + jnp.full((16,), i * 8 + u * 2, jnp.int32)
                rec = plsc.load_gather(buf_in, [src])
                buf_out[pl.ds(i * 64 + u * 16, 16)] = rec
            return carry

        lax.fori_loop(0, nfields * per_w // 64, body, 0)
        pltpu.sync_copy(buf_out, aos_hbm.at[pl.ds(g0 * nfields,
                                                  nfields * per_w)])

    return sc_repack


def _blend_body(k_hits, a_ref, b_ref, am_ref):
    at = jnp.clip(a_ref[...].T, 0.0, 0.999)
    rows = at.shape[1]
    trans = jnp.ones((1, rows), jnp.float32)
    bls = []
    for k in range(k_hits):
        ak = at[k:k + 1, :]
        bls.append(trans * ak)
        trans = trans * (1.0 - ak)
    blend = jnp.concatenate(bls, axis=0)
    b_ref[...] = blend
    am_ref[...] = jnp.sum(blend, axis=0, keepdims=True)[None]


def _blend_tc(alpha, block_px=2048):
    p, k_hits = alpha.shape
    grid = p // block_px
    body = functools.partial(_blend_body, k_hits)
    return pl.pallas_call(
        body,
        grid=(grid,),
        in_specs=[pl.BlockSpec((block_px, k_hits), lambda i: (i, 0))],
        out_specs=[
            pl.BlockSpec((k_hits, block_px), lambda i: (0, i)),
            pl.BlockSpec((1, 1, block_px), lambda i: (i, 0, 0)),
        ],
        out_shape=[
            jax.ShapeDtypeStruct((k_hits, p), jnp.float32),
            jax.ShapeDtypeStruct((grid, 1, block_px), jnp.float32),
        ],
    )(alpha)


def _make_sc_reduce(n_pad, total_px, cb_dim, k_hits):
    info = plsc.get_sparse_core_info()
    nc = info.num_cores
    nw = nc * info.num_subcores
    chunk_px = 256
    chunk_slots = chunk_px * k_hits
    px_per_w = total_px // nw
    nchunk = px_per_w // chunk_px
    idx_rows = chunk_slots // 128
    wm_words = chunk_px * cb_dim
    mesh = plsc.VectorSubcoreMesh(core_axis_name="c", subcore_axis_name="s")

    @functools.partial(
        pl.kernel,
        out_type=jax.ShapeDtypeStruct((total_px * cb_dim,), jnp.float32),
        mesh=mesh,
        compiler_params=_SC_PARAMS,
        scratch_types=[
            pltpu.VMEM((2, idx_rows, 128), jnp.int32),
            pltpu.VMEM((2, chunk_slots, k_hits), jnp.float32),
            pltpu.VMEM((2, k_hits * chunk_px), jnp.float32),
            pltpu.VMEM((wm_words,), jnp.float32),
            pltpu.SemaphoreType.DMA,
            pltpu.SemaphoreType.DMA,
        ],
    )
    def sc_reduce(aos_hbm, idx_hbm, blt_hbm, out_hbm,
                  idx_v, pk_v, bl_v, wm_v, sem, sem_out):
        wid = lax.axis_index("s") * nc + lax.axis_index("c")
        lanes = lax.broadcasted_iota(jnp.int32, (16,), 0)
        rowpat = lanes // 4
        colpat = lanes % 4
        blpat = rowpat * chunk_px
        zeros16 = jnp.zeros((16,), jnp.float32)
        unroll = 4

        def prefetch(c):
            b = c % 2
            r0 = wid * (idx_rows * nchunk) + c * idx_rows
            p0 = wid * px_per_w + c * chunk_px
            pltpu.sync_copy(idx_hbm.at[pl.ds(r0, idx_rows)], idx_v.at[b])
            cps = []
            for j in range(idx_rows):
                cps.append(pltpu.async_copy(
                    aos_hbm.at[idx_v.at[b, j]],
                    pk_v.at[b, pl.ds(j * 128, 128)], sem))
            for k in range(k_hits):
                cps.append(pltpu.async_copy(
                    blt_hbm.at[pl.ds(k * total_px + p0, chunk_px)],
                    bl_v.at[b, pl.ds(k * chunk_px, chunk_px)], sem))
            return cps

        copies = prefetch(0)
        prev_out = None
        for c in range(nchunk):
            b = c % 2
            w0 = wid * (wm_words * nchunk) + c * wm_words
            if prev_out is not None:
                prev_out.wait()

            def zero_body(i, zc):
                for t in range(8):
                    wm_v[pl.ds(i * 128 + t * 16, 16)] = zeros16
                return zc
            lax.fori_loop(0, wm_words // 128, zero_body, 0)
            for cp in copies:
                cp.wait()
            if c + 1 < nchunk:
                copies = prefetch(c + 1)

            def px_body(i, pc):
                for u in range(unroll):
                    p = i * unroll + u
                    base = jnp.full((16,), p * cb_dim, jnp.int32)
                    sp = jnp.full((16,), p * k_hits, jnp.int32) + rowpat
                    blp = jnp.full((16,), p, jnp.int32) + blpat
                    for half in range(2):
                        rows = sp + (half * 4)
                        vals = plsc.load_gather(pk_v.at[b], [rows, colpat])
                        idxf = plsc.load_gather(pk_v.at[b],
                                                [rows, colpat + 4])
                        bl = plsc.load_gather(
                            bl_v.at[b], [blp + (half * 4 * chunk_px)])
                        tgt = idxf.astype(jnp.int32) + base
                        plsc.addupdate_scatter(wm_v, [tgt], vals * bl)
                return pc
            lax.fori_loop(0, chunk_px // unroll, px_body, 0)

            prev_out = pltpu.async_copy(
                wm_v, out_hbm.at[pl.ds(w0, wm_words)], sem_out)
        prev_out.wait()

    return sc_reduce


def _decode_body(w_ref, c_ref, f_ref):
    f_ref[...] = jnp.dot(w_ref[...], c_ref[...],
                         preferred_element_type=jnp.float32)


def _decode_matmul(wm, codebook, block_px=512):
    p, cb_dim = wm.shape
    clip_dims = codebook.shape[1]
    grid = p // block_px
    return pl.pallas_call(
        _decode_body,
        grid=(grid,),
        in_specs=[
            pl.BlockSpec((block_px, cb_dim), lambda i: (i, 0)),
            pl.BlockSpec((cb_dim, clip_dims), lambda i: (0, 0)),
        ],
        out_specs=pl.BlockSpec((block_px, clip_dims), lambda i: (i, 0)),
        out_shape=jax.ShapeDtypeStruct((p, clip_dims), jnp.float32),
    )(wm, codebook)


def kernel(world_to_camera, projection, image_width, image_height,
           pixel_gaussian_idx, pixel_alpha, logits, codebooks):
    n, cb_dim = logits.shape
    bz, h, w, k_hits = pixel_alpha.shape
    clip_dims = codebooks.shape[2]
    p = bz * h * w
    total_slots = p * k_hits
    nfields = 2 * TOPK

    idx2d = pixel_gaussian_idx.reshape(total_slots // 128, 128).astype(jnp.int32)
    alpha = pixel_alpha.reshape(p, k_hits)

    fm = _topk_pack(logits)
    n_pad = fm.shape[1]
    aos1d = _make_sc_repack(n_pad, nfields)(fm.reshape(nfields * n_pad))
    blt, alpha_map = _blend_tc(alpha)
    wm1d = _make_sc_reduce(n_pad, p, cb_dim, k_hits)(
        aos1d.reshape(n_pad, nfields), idx2d, blt.reshape(k_hits * p))
    feature = _decode_matmul(wm1d.reshape(p, cb_dim), codebooks[0])
    return (feature.reshape(bz, h, w, clip_dims),
            alpha_map.reshape(bz, h, w, 1))


# speedup vs baseline: 33.3010x; 1.2590x over previous
import functools

import jax
import jax.numpy as jnp
from jax import lax
from jax.experimental import pallas as pl
from jax.experimental.pallas import tpu as pltpu
from jax.experimental.pallas import tpu_sc as plsc

TOPK = 4

_SC_PARAMS = pltpu.CompilerParams(use_tc_tiling_on_sc=False,
                                  needs_layout_passes=False)


def _topk_pack_body(x_ref, o_ref):
    xt = x_ref[...]
    cols, rows = xt.shape
    iota = lax.broadcasted_iota(jnp.int32, (cols, rows), 0)
    xb = lax.bitcast_convert_type(xt, jnp.int32)
    tie = jnp.where(xb >= 0, (cols - 1) - iota, iota)
    key = lax.bitcast_convert_type((xb & ~63) | tie, jnp.float32)
    ms = []
    for _ in range(TOPK):
        m = jnp.max(key, axis=0, keepdims=True)
        key = jnp.where(key == m, -jnp.inf, key)
        ms.append(m)
    m_cat = jnp.concatenate(ms, axis=0)
    mb = lax.bitcast_convert_type(m_cat, jnp.int32)
    low = mb & 63
    lanes = jnp.where(mb >= 0, (cols - 1) - low, low)
    e = jnp.exp(m_cat - ms[0])
    soft = e / jnp.sum(e, axis=0, keepdims=True)
    o_ref[...] = jnp.concatenate([soft, lanes.astype(jnp.float32)], axis=0)


def _topk_pack(logits_t, block_rows=2048):
    c, n = logits_t.shape
    grid = -(-n // block_rows)
    n_pad = grid * block_rows
    return pl.pallas_call(
        _topk_pack_body,
        grid=(grid,),
        in_specs=[pl.BlockSpec((c, block_rows), lambda i: (0, i))],
        out_specs=pl.BlockSpec((2 * TOPK, block_rows), lambda i: (0, i)),
        out_shape=jax.ShapeDtypeStruct((2 * TOPK, n_pad), jnp.float32),
    )(logits_t)


def _make_sc_repack(n_pad, nfields):
    info = plsc.get_sparse_core_info()
    nw = info.num_cores * info.num_subcores
    per_w = n_pad // nw
    mesh = plsc.VectorSubcoreMesh(core_axis_name="c", subcore_axis_name="s")

    @functools.partial(
        pl.kernel,
        out_type=jax.ShapeDtypeStruct((n_pad * nfields,), jnp.float32),
        mesh=mesh,
        compiler_params=_SC_PARAMS,
        scratch_types=[
            pltpu.VMEM((nfields * per_w,), jnp.float32),
            pltpu.VMEM((nfields * per_w,), jnp.float32),
        ],
    )
    def sc_repack(fm_hbm, aos_hbm, buf_in, buf_out):
        wid = lax.axis_index("s") * info.num_cores + lax.axis_index("c")
        g0 = wid * per_w
        for f in range(nfields):
            pltpu.sync_copy(fm_hbm.at[pl.ds(f * n_pad + g0, per_w)],
                            buf_in.at[pl.ds(f * per_w, per_w)])

        lanes = lax.broadcasted_iota(jnp.int32, (16,), 0)
        base_src = (lanes % nfields) * per_w + lanes // nfields

        def body(i, carry):
            for u in range(4):
                src = base_src + jnp.full((16,), i * 8 + u * 2, jnp.int32)
                rec = plsc.load_gather(buf_in, [src])
                buf_out[pl.ds(i * 64 + u * 16, 16)] = rec
            return carry

        lax.fori_loop(0, nfields * per_w // 64, body, 0)
        pltpu.sync_copy(buf_out, aos_hbm.at[pl.ds(g0 * nfields,
                                                  nfields * per_w)])

    return sc_repack


def _blend_body(k_hits, a_ref, b_ref, am_ref):
    at = jnp.clip(a_ref[...].T, 0.0, 0.999)
    rows = at.shape[1]
    trans = jnp.ones((1, rows), jnp.float32)
    bls = []
    for k in range(k_hits):
        ak = at[k:k + 1, :]
        bls.append(trans * ak)
        trans = trans * (1.0 - ak)
    blend = jnp.concatenate(bls, axis=0)
    b_ref[...] = blend
    am_ref[...] = jnp.sum(blend, axis=0, keepdims=True)[None]


def _blend_tc(alpha, block_px=2048):
    p, k_hits = alpha.shape
    grid = p // block_px
    body = functools.partial(_blend_body, k_hits)
    return pl.pallas_call(
        body,
        grid=(grid,),
        in_specs=[pl.BlockSpec((block_px, k_hits), lambda i: (i, 0))],
        out_specs=[
            pl.BlockSpec((k_hits, block_px), lambda i: (0, i)),
            pl.BlockSpec((1, 1, block_px), lambda i: (i, 0, 0)),
        ],
        out_shape=[
            jax.ShapeDtypeStruct((k_hits, p), jnp.float32),
            jax.ShapeDtypeStruct((grid, 1, block_px), jnp.float32),
        ],
    )(alpha)


def _make_sc_reduce(n_pad, total_px, cb_dim, k_hits):
    info = plsc.get_sparse_core_info()
    nc = info.num_cores
    nw = nc * info.num_subcores
    chunk_px = 256
    chunk_slots = chunk_px * k_hits
    px_per_w = total_px // nw
    nchunk = px_per_w // chunk_px
    idx_rows = chunk_slots // 128
    wm_words = chunk_px * cb_dim
    mesh = plsc.VectorSubcoreMesh(core_axis_name="c", subcore_axis_name="s")

    @functools.partial(
        pl.kernel,
        out_type=jax.ShapeDtypeStruct((total_px, 128), jnp.float32),
        mesh=mesh,
        compiler_params=_SC_PARAMS,
        scratch_types=[
            pltpu.VMEM((2, idx_rows, 128), jnp.int32),
            pltpu.VMEM((2, chunk_slots, k_hits), jnp.float32),
            pltpu.VMEM((2, k_hits * chunk_px), jnp.float32),
            pltpu.VMEM((chunk_px, cb_dim), jnp.float32),
            pltpu.SemaphoreType.DMA,
            pltpu.SemaphoreType.DMA,
        ],
    )
    def sc_reduce(aos_hbm, idx_hbm, blt_hbm, out_hbm,
                  idx_v, pk_v, bl_v, wm_v, sem, sem_out):
        wid = lax.axis_index("s") * nc + lax.axis_index("c")
        lanes = lax.broadcasted_iota(jnp.int32, (16,), 0)
        rowpat = lanes // 4
        colpat = lanes % 4
        blpat = rowpat * chunk_px
        zeros16 = jnp.zeros((16,), jnp.float32)
        unroll = 4

        def prefetch(c):
            b = c % 2
            r0 = wid * (idx_rows * nchunk) + c * idx_rows
            p0 = wid * px_per_w + c * chunk_px
            pltpu.sync_copy(idx_hbm.at[pl.ds(r0, idx_rows)], idx_v.at[b])
            cps = []
            for j in range(idx_rows):
                cps.append(pltpu.async_copy(
                    aos_hbm.at[idx_v.at[b, j]],
                    pk_v.at[b, pl.ds(j * 128, 128)], sem))
            for k in range(k_hits):
                cps.append(pltpu.async_copy(
                    blt_hbm.at[pl.ds(k * total_px + p0, chunk_px)],
                    bl_v.at[b, pl.ds(k * chunk_px, chunk_px)], sem))
            return cps

        copies = prefetch(0)
        prev_out = None
        for c in range(nchunk):
            b = c % 2
            p0 = wid * px_per_w + c * chunk_px
            if prev_out is not None:
                prev_out.wait()

            def zero_body(i, zc):
                for t in range(2):
                    for j in range(cb_dim // 16):
                        wm_v[i * 2 + t, pl.ds(j * 16, 16)] = zeros16
                return zc
            lax.fori_loop(0, chunk_px // 2, zero_body, 0)
            for cp in copies:
                cp.wait()
            if c + 1 < nchunk:
                copies = prefetch(c + 1)

            def px_body(i, pc):
                for u in range(unroll):
                    p = i * unroll + u
                    prow = jnp.full((16,), p, jnp.int32)
                    sp = jnp.full((16,), p * k_hits, jnp.int32) + rowpat
                    blp = prow + blpat
                    for half in range(2):
                        rows = sp + (half * 4)
                        vals = plsc.load_gather(pk_v.at[b], [rows, colpat])
                        idxf = plsc.load_gather(pk_v.at[b],
                                                [rows, colpat + 4])
                        bl = plsc.load_gather(
                            bl_v.at[b], [blp + (half * 4 * chunk_px)])
                        plsc.addupdate_scatter(
                            wm_v, [prow, idxf.astype(jnp.int32)], vals * bl)
                return pc
            lax.fori_loop(0, chunk_px // unroll, px_body, 0)

            prev_out = pltpu.async_copy(
                wm_v, out_hbm.at[pl.ds(p0, chunk_px), pl.ds(0, cb_dim)],
                sem_out)
        prev_out.wait()

    return sc_reduce


def _decode_body(cb_dim, w_ref, c_ref, f_ref):
    f_ref[...] = jnp.dot(w_ref[...][:, :cb_dim], c_ref[...],
                         preferred_element_type=jnp.float32)


def _decode_matmul(wm, codebook, block_px=512):
    p = wm.shape[0]
    cb_dim, clip_dims = codebook.shape
    grid = p // block_px
    return pl.pallas_call(
        functools.partial(_decode_body, cb_dim),
        grid=(grid,),
        in_specs=[
            pl.BlockSpec((block_px, 128), lambda i: (i, 0)),
            pl.BlockSpec((cb_dim, clip_dims), lambda i: (0, 0)),
        ],
        out_specs=pl.BlockSpec((block_px, clip_dims), lambda i: (i, 0)),
        out_shape=jax.ShapeDtypeStruct((p, clip_dims), jnp.float32),
    )(wm, codebook)


def kernel(world_to_camera, projection, image_width, image_height,
           pixel_gaussian_idx, pixel_alpha, logits, codebooks):
    n, cb_dim = logits.shape
    bz, h, w, k_hits = pixel_alpha.shape
    clip_dims = codebooks.shape[2]
    p = bz * h * w
    total_slots = p * k_hits
    nfields = 2 * TOPK

    idx2d = pixel_gaussian_idx.reshape(total_slots // 128, 128).astype(jnp.int32)
    alpha = pixel_alpha.reshape(p, k_hits)

    fm = _topk_pack(logits.T)
    n_pad = fm.shape[1]
    aos1d = _make_sc_repack(n_pad, nfields)(fm.reshape(nfields * n_pad))
    blt, alpha_map = _blend_tc(alpha)
    wm2d = _make_sc_reduce(n_pad, p, cb_dim, k_hits)(
        aos1d.reshape(n_pad, nfields), idx2d, blt.reshape(k_hits * p))
    feature = _decode_matmul(wm2d, codebooks[0])
    return (feature.reshape(bz, h, w, clip_dims),
            alpha_map.reshape(bz, h, w, 1))


# speedup vs baseline: 39.6951x; 1.1920x over previous
import functools

import jax
import jax.numpy as jnp
from jax import lax
from jax.experimental import pallas as pl
from jax.experimental.pallas import tpu as pltpu
from jax.experimental.pallas import tpu_sc as plsc

TOPK = 4

_SC_PARAMS = pltpu.CompilerParams(use_tc_tiling_on_sc=False,
                                  needs_layout_passes=False)


def _topk_pack_body(x_ref, o_ref):
    xt = x_ref[...]
    cols, rows = xt.shape
    iota = lax.broadcasted_iota(jnp.int32, (cols, rows), 0)
    xb = lax.bitcast_convert_type(xt, jnp.int32)
    tie = jnp.where(xb >= 0, (cols - 1) - iota, iota)
    key = lax.bitcast_convert_type((xb & ~63) | tie, jnp.float32)
    ms = []
    for _ in range(TOPK):
        m = jnp.max(key, axis=0, keepdims=True)
        key = jnp.where(key == m, -jnp.inf, key)
        ms.append(m)
    m_cat = jnp.concatenate(ms, axis=0)
    mb = lax.bitcast_convert_type(m_cat, jnp.int32)
    low = mb & 63
    lanes = jnp.where(mb >= 0, (cols - 1) - low, low)
    e = jnp.exp(m_cat - ms[0])
    soft = e / jnp.sum(e, axis=0, keepdims=True)
    o_ref[...] = jnp.concatenate([soft, lanes.astype(jnp.float32)], axis=0)


def _topk_pack(logits_t, block_rows=2048):
    c, n = logits_t.shape
    grid = -(-n // block_rows)
    n_pad = grid * block_rows
    return pl.pallas_call(
        _topk_pack_body,
        grid=(grid,),
        in_specs=[pl.BlockSpec((c, block_rows), lambda i: (0, i))],
        out_specs=pl.BlockSpec((2 * TOPK, block_rows), lambda i: (0, i)),
        out_shape=jax.ShapeDtypeStruct((2 * TOPK, n_pad), jnp.float32),
    )(logits_t)


def _make_sc_repack(n_pad, nfields):
    info = plsc.get_sparse_core_info()
    nw = info.num_cores * info.num_subcores
    per_w = n_pad // nw
    mesh = plsc.VectorSubcoreMesh(core_axis_name="c", subcore_axis_name="s")

    @functools.partial(
        pl.kernel,
        out_type=jax.ShapeDtypeStruct((n_pad * nfields,), jnp.float32),
        mesh=mesh,
        compiler_params=_SC_PARAMS,
        scratch_types=[
            pltpu.VMEM((nfields * per_w,), jnp.float32),
            pltpu.VMEM((nfields * per_w,), jnp.float32),
        ],
    )
    def sc_repack(fm_hbm, aos_hbm, buf_in, buf_out):
        wid = lax.axis_index("s") * info.num_cores + lax.axis_index("c")
        g0 = wid * per_w
        for f in range(nfields):
            pltpu.sync_copy(fm_hbm.at[pl.ds(f * n_pad + g0, per_w)],
                            buf_in.at[pl.ds(f * per_w, per_w)])

        lanes = lax.broadcasted_iota(jnp.int32, (16,), 0)
        base_src = (lanes % nfields) * per_w + lanes // nfields

        def body(i, carry):
            for u in range(4):
                src = base_src + jnp.full((16,), i * 8 + u * 2, jnp.int32)
                rec = plsc.load_gather(buf_in, [src])
                buf_out[pl.ds(i * 64 + u * 16, 16)] = rec
            return carry

        lax.fori_loop(0, nfields * per_w // 64, body, 0)
        pltpu.sync_copy(buf_out, aos_hbm.at[pl.ds(g0 * nfields,
                                                  nfields * per_w)])

    return sc_repack


def _blend_body(k_hits, a_ref, b_ref, am_ref):
    at = jnp.clip(a_ref[...].T, 0.0, 0.999)
    rows = at.shape[1]
    trans = jnp.ones((1, rows), jnp.float32)
    bls = []
    for k in range(k_hits):
        ak = at[k:k + 1, :]
        bls.append(trans * ak)
        trans = trans * (1.0 - ak)
    blend = jnp.concatenate(bls, axis=0)
    b_ref[...] = blend
    am_ref[...] = jnp.sum(blend, axis=0, keepdims=True)[None]


def _blend_tc(alpha, block_px=2048):
    p, k_hits = alpha.shape
    grid = p // block_px
    body = functools.partial(_blend_body, k_hits)
    return pl.pallas_call(
        body,
        grid=(grid,),
        in_specs=[pl.BlockSpec((block_px, k_hits), lambda i: (i, 0))],
        out_specs=[
            pl.BlockSpec((k_hits, block_px), lambda i: (0, i)),
            pl.BlockSpec((1, 1, block_px), lambda i: (i, 0, 0)),
        ],
        out_shape=[
            jax.ShapeDtypeStruct((k_hits, p), jnp.float32),
            jax.ShapeDtypeStruct((grid, 1, block_px), jnp.float32),
        ],
    )(alpha)


def _make_sc_reduce(n_pad, total_px, cb_dim, k_hits):
    info = plsc.get_sparse_core_info()
    nc = info.num_cores
    nw = nc * info.num_subcores
    chunk_px = 256
    chunk_slots = chunk_px * k_hits
    px_per_w = total_px // nw
    nchunk = px_per_w // chunk_px
    idx_rows = chunk_slots // 128
    wm_words = chunk_px * cb_dim
    mesh = plsc.VectorSubcoreMesh(core_axis_name="c", subcore_axis_name="s")

    @functools.partial(
        pl.kernel,
        out_type=jax.ShapeDtypeStruct((total_px, 128), jnp.float32),
        mesh=mesh,
        compiler_params=_SC_PARAMS,
        scratch_types=[
            pltpu.VMEM((2, idx_rows, 128), jnp.int32),
            pltpu.VMEM((2, chunk_slots, k_hits), jnp.float32),
            pltpu.VMEM((2, k_hits * chunk_px), jnp.float32),
            pltpu.VMEM((chunk_px, cb_dim), jnp.float32),
            pltpu.SemaphoreType.DMA,
            pltpu.SemaphoreType.DMA,
        ],
    )
    def sc_reduce(aos_hbm, idx_hbm, blt_hbm, out_hbm,
                  idx_v, pk_v, bl_v, wm_v, sem, sem_out):
        wid = lax.axis_index("s") * nc + lax.axis_index("c")
        lanes = lax.broadcasted_iota(jnp.int32, (16,), 0)
        rowpat = lanes // 4
        colpat = lanes % 4
        blpat = rowpat * chunk_px
        zeros16 = jnp.zeros((16,), jnp.float32)
        unroll = 4

        def prefetch(c):
            b = c % 2
            r0 = wid * (idx_rows * nchunk) + c * idx_rows
            p0 = wid * px_per_w + c * chunk_px
            pltpu.sync_copy(idx_hbm.at[pl.ds(r0, idx_rows)], idx_v.at[b])
            cps = []
            for j in range(idx_rows):
                cps.append(pltpu.async_copy(
                    aos_hbm.at[idx_v.at[b, j]],
                    pk_v.at[b, pl.ds(j * 128, 128)], sem))
            for k in range(k_hits):
                cps.append(pltpu.async_copy(
                    blt_hbm.at[pl.ds(k * total_px + p0, chunk_px)],
                    bl_v.at[b, pl.ds(k * chunk_px, chunk_px)], sem))
            return cps

        copies = prefetch(0)
        prev_out = None
        for c in range(nchunk):
            b = c % 2
            p0 = wid * px_per_w + c * chunk_px
            if prev_out is not None:
                prev_out.wait()

            def zero_body(i, zc):
                for t in range(2):
                    for j in range(cb_dim // 16):
                        wm_v[i * 2 + t, pl.ds(j * 16, 16)] = zeros16
                return zc
            lax.fori_loop(0, chunk_px // 2, zero_body, 0)
            for cp in copies:
                cp.wait()
            if c + 1 < nchunk:
                copies = prefetch(c + 1)

            @functools.partial(plsc.parallel_loop, 0, chunk_px // unroll)
            def px_body(i):
                for u in range(unroll):
                    p = i * unroll + u
                    prow = jnp.full((16,), p, jnp.int32)
                    sp = jnp.full((16,), p * k_hits, jnp.int32) + rowpat
                    blp = prow + blpat
                    for half in range(2):
                        rows = sp + (half * 4)
                        vals = plsc.load_gather(pk_v.at[b], [rows, colpat])
                        idxf = plsc.load_gather(pk_v.at[b],
                                                [rows, colpat + 4])
                        bl = plsc.load_gather(
                            bl_v.at[b], [blp + (half * 4 * chunk_px)])
                        plsc.addupdate_scatter(
                            wm_v, [prow, idxf.astype(jnp.int32)], vals * bl)

            prev_out = pltpu.async_copy(
                wm_v, out_hbm.at[pl.ds(p0, chunk_px), pl.ds(0, cb_dim)],
                sem_out)
        prev_out.wait()

    return sc_reduce


def _decode_body(cb_dim, w_ref, c_ref, f_ref):
    f_ref[...] = jnp.dot(w_ref[...][:, :cb_dim], c_ref[...],
                         preferred_element_type=jnp.float32)


def _decode_matmul(wm, codebook, block_px=512):
    p = wm.shape[0]
    cb_dim, clip_dims = codebook.shape
    grid = p // block_px
    return pl.pallas_call(
        functools.partial(_decode_body, cb_dim),
        grid=(grid,),
        in_specs=[
            pl.BlockSpec((block_px, 128), lambda i: (i, 0)),
            pl.BlockSpec((cb_dim, clip_dims), lambda i: (0, 0)),
        ],
        out_specs=pl.BlockSpec((block_px, clip_dims), lambda i: (i, 0)),
        out_shape=jax.ShapeDtypeStruct((p, clip_dims), jnp.float32),
    )(wm, codebook)


def kernel(world_to_camera, projection, image_width, image_height,
           pixel_gaussian_idx, pixel_alpha, logits, codebooks):
    n, cb_dim = logits.shape
    bz, h, w, k_hits = pixel_alpha.shape
    clip_dims = codebooks.shape[2]
    p = bz * h * w
    total_slots = p * k_hits
    nfields = 2 * TOPK

    idx2d = pixel_gaussian_idx.reshape(total_slots // 128, 128).astype(jnp.int32)
    alpha = pixel_alpha.reshape(p, k_hits)

    fm = _topk_pack(logits.T)
    n_pad = fm.shape[1]
    aos1d = _make_sc_repack(n_pad, nfields)(fm.reshape(nfields * n_pad))
    blt, alpha_map = _blend_tc(alpha)
    wm2d = _make_sc_reduce(n_pad, p, cb_dim, k_hits)(
        aos1d.reshape(n_pad, nfields), idx2d, blt.reshape(k_hits * p))
    feature = _decode_matmul(wm2d, codebooks[0])
    return (feature.reshape(bz, h, w, clip_dims),
            alpha_map.reshape(bz, h, w, 1))
